# Initial kernel scaffold; baseline (speedup 1.0000x reference)
import jax
import jax.numpy as jnp
from jax.experimental import pallas as pl


def kernel(input_, edge, W, a):
    raise NotImplementedError("write your pallas kernel here")


# speedup vs baseline: 6.7800x; 6.7800x over previous
import jax
import jax.numpy as jnp
from jax import lax
from jax.experimental import pallas as pl
from jax.experimental.pallas import tpu as pltpu
from jax.experimental.pallas import tpu_sc as plsc

N = 10000
D = 128
E = 320000
NEG_SLOPE = 0.2
NC, NS, L = 2, 16, 16
NW = NC * NS
EPW = E // NW
CHUNK = 80
NCHUNK = EPW // CHUNK
COVER = 640
LASTN0 = N - COVER


def _mm_body(x_ref, w_ref, a_ref, h1_ref, h2_ref, s1_ref, s2_ref):
    x = x_ref[...]
    w = w_ref[...]
    av = a_ref[...]
    dn = (((1,), (1,)), ((), ()))
    h1 = lax.dot_general(x, w[:, :D], dn, preferred_element_type=jnp.float32)
    h2 = lax.dot_general(x, w[:, D:], dn, preferred_element_type=jnp.float32)
    h1_ref[...] = h1
    h2_ref[...] = h2
    s1_ref[...] = lax.dot_general(h1, av, dn, preferred_element_type=jnp.float32)
    s2_ref[...] = lax.dot_general(h2, av, dn, preferred_element_type=jnp.float32)


_mm_call = pl.pallas_call(
    _mm_body,
    out_shape=[
        jax.ShapeDtypeStruct((N, D), jnp.float32),
        jax.ShapeDtypeStruct((N, D), jnp.float32),
        jax.ShapeDtypeStruct((N, 1), jnp.float32),
        jax.ShapeDtypeStruct((N, 1), jnp.float32),
    ],
)


def _sc_body(e0_hbm, e1_hbm, s1_hbm, s2_hbm, h2_hbm, part_hbm, rsum_hbm,
             e0_v, e1_v, g1_v, g2_v, w_v, rows_v, acc_sh, rs_sh,
             sem1, sem2, sem3):
    cid = lax.axis_index("c")
    sid = lax.axis_index("s")
    wid = cid * NS + sid

    zrow0 = jnp.zeros((L,), jnp.float32)

    def zrow(r, carry):
        for j in range(D // L):
            rows_v[r, pl.ds(j * L, L)] = zrow0
        return carry

    lax.fori_loop(0, CHUNK, zrow, 0)
    for j in range(CHUNK // L):
        w_v[pl.ds(j * L, L)] = zrow0
    n0 = jnp.minimum(sid * COVER, LASTN0)
    for k in range(COVER // CHUNK):
        pltpu.sync_copy(rows_v, acc_sh.at[pl.ds(n0 + k * CHUNK, CHUNK)])
        pltpu.sync_copy(w_v, rs_sh.at[pl.ds(n0 + k * CHUNK, CHUNK)])
    plsc.subcore_barrier()

    def chunk_body(c, carry):
        base = wid * EPW + c * CHUNK
        pltpu.sync_copy(e0_hbm.at[pl.ds(base, CHUNK)], e0_v)
        pltpu.sync_copy(e1_hbm.at[pl.ds(base, CHUNK)], e1_v)
        cp1 = pltpu.async_copy(s1_hbm.at[e0_v], g1_v, sem1)
        cp2 = pltpu.async_copy(s2_hbm.at[e1_v], g2_v, sem2)
        cp3 = pltpu.async_copy(h2_hbm.at[e1_v], rows_v, sem3)
        cp1.wait()
        cp2.wait()
        for j in range(CHUNK // L):
            x = g1_v[pl.ds(j * L, L)] + g2_v[pl.ds(j * L, L)]
            w_v[pl.ds(j * L, L)] = jnp.exp(-jnp.maximum(x, NEG_SLOPE * x))
        cp3.wait()

        def srow(g, carry2):
            wgrp = w_v[pl.ds(g * L, L)]
            for u in range(L):
                i = g * L + u
                wv = wgrp[u]
                for j in range(D // L):
                    rows_v[i, pl.ds(j * L, L)] = rows_v[i, pl.ds(j * L, L)] * wv
            return carry2

        lax.fori_loop(0, CHUNK // L, srow, 0)
        pltpu.sync_copy(rows_v, acc_sh.at[e0_v], add=True)
        pltpu.sync_copy(w_v, rs_sh.at[e0_v], add=True)
        return carry

    lax.fori_loop(0, NCHUNK, chunk_body, 0)
    plsc.subcore_barrier()

    pltpu.sync_copy(acc_sh.at[pl.ds(n0, COVER)],
                    part_hbm.at[cid, pl.ds(n0, COVER)])
    pltpu.sync_copy(rs_sh.at[pl.ds(n0, COVER)],
                    rsum_hbm.at[cid, pl.ds(n0, COVER)])


_sc_call = pl.kernel(
    _sc_body,
    out_type=(
        jax.ShapeDtypeStruct((NC, N, D), jnp.float32),
        jax.ShapeDtypeStruct((NC, N), jnp.float32),
    ),
    mesh=plsc.VectorSubcoreMesh(core_axis_name="c", subcore_axis_name="s",
                                num_cores=NC, num_subcores=NS),
    compiler_params=pltpu.CompilerParams(use_tc_tiling_on_sc=False,
                                         needs_layout_passes=False),
    scratch_types=[
        pltpu.VMEM((CHUNK,), jnp.int32),
        pltpu.VMEM((CHUNK,), jnp.int32),
        pltpu.VMEM((CHUNK,), jnp.float32),
        pltpu.VMEM((CHUNK,), jnp.float32),
        pltpu.VMEM((CHUNK,), jnp.float32),
        pltpu.VMEM((CHUNK, D), jnp.float32),
        pltpu.VMEM_SHARED((N, D), jnp.float32),
        pltpu.VMEM_SHARED((N,), jnp.float32),
        pltpu.SemaphoreType.DMA,
        pltpu.SemaphoreType.DMA,
        pltpu.SemaphoreType.DMA,
    ],
)


def _comb_body(h1_ref, part_ref, rsum_ref, o_ref):
    acc = part_ref[0] + part_ref[1]
    rs = rsum_ref[0] + rsum_ref[1]
    denom = jnp.where(rs == 0.0, 1e-12, rs)
    h = (h1_ref[...] * rs + acc) / denom
    o_ref[...] = jnp.where(h > 0.0, h, jnp.exp(h) - 1.0)


_comb_call = pl.pallas_call(
    _comb_body,
    out_shape=jax.ShapeDtypeStruct((N, D), jnp.float32),
)


def kernel(input_, edge, W, a):
    edge = edge.astype(jnp.int32)
    h1, h2, s1, s2 = _mm_call(input_, W, a)
    part, rsum = _sc_call(edge[0], edge[1], s1.reshape(N), s2.reshape(N), h2)
    return _comb_call(h1, part, rsum.reshape(NC, N, 1))


# speedup vs baseline: 11.8107x; 1.7420x over previous
import jax
import jax.numpy as jnp
from jax import lax
from jax.experimental import pallas as pl
from jax.experimental.pallas import tpu as pltpu
from jax.experimental.pallas import tpu_sc as plsc

N = 10000
D = 128
E = 320000
NEG_SLOPE = 0.2
NC, NS, L = 2, 16, 16
NW = NC * NS
EPW = E // NW
CHUNK = 80
NCHUNK = EPW // CHUNK
COVER = 640
LASTN0 = N - COVER


def _mm_body(x_ref, w_ref, a_ref, h1_ref, h2_ref, s1_ref, s2_ref):
    x = x_ref[...]
    w = w_ref[...]
    av = a_ref[...]
    dn = (((1,), (1,)), ((), ()))
    h1 = lax.dot_general(x, w[:, :D], dn, preferred_element_type=jnp.float32)
    h2 = lax.dot_general(x, w[:, D:], dn, preferred_element_type=jnp.float32)
    h1_ref[...] = h1
    h2_ref[...] = h2
    s1_ref[...] = lax.dot_general(h1, av, dn, preferred_element_type=jnp.float32)
    s2_ref[...] = lax.dot_general(h2, av, dn, preferred_element_type=jnp.float32)


_mm_call = pl.pallas_call(
    _mm_body,
    out_shape=[
        jax.ShapeDtypeStruct((N, D), jnp.float32),
        jax.ShapeDtypeStruct((N, D), jnp.float32),
        jax.ShapeDtypeStruct((N, 1), jnp.float32),
        jax.ShapeDtypeStruct((N, 1), jnp.float32),
    ],
)


def _sc_body(e0_hbm, e1_hbm, s1_hbm, s2_hbm, h2_hbm, part_hbm, rsum_hbm,
             e0f_v, e1f_v, g1a_v, g1b_v, g2a_v, g2b_v, wa_v, wb_v,
             rowsa_v, rowsb_v, acc_sh, rs_sh, semga, semgb, semsa, semsb):
    cid = lax.axis_index("c")
    sid = lax.axis_index("s")
    wid = cid * NS + sid
    base = wid * EPW

    pltpu.sync_copy(e0_hbm.at[pl.ds(base, EPW)], e0f_v)
    pltpu.sync_copy(e1_hbm.at[pl.ds(base, EPW)], e1f_v)

    zrow0 = jnp.zeros((L,), jnp.float32)

    def zrow(r, carry):
        for j in range(D // L):
            rowsa_v[r, pl.ds(j * L, L)] = zrow0
            rowsb_v[r, pl.ds(j * L, L)] = zrow0
        return carry

    lax.fori_loop(0, CHUNK, zrow, 0)
    for j in range(CHUNK // L):
        wa_v[pl.ds(j * L, L)] = zrow0
        wb_v[pl.ds(j * L, L)] = zrow0
    n0 = jnp.minimum(sid * COVER, LASTN0)
    for k in range(COVER // CHUNK):
        pltpu.sync_copy(rowsa_v, acc_sh.at[pl.ds(n0 + k * CHUNK, CHUNK)])
        pltpu.sync_copy(wa_v, rs_sh.at[pl.ds(n0 + k * CHUNK, CHUNK)])
    plsc.subcore_barrier()

    bufs = {
        0: (g1a_v, g2a_v, wa_v, rowsa_v, semga, semsa),
        1: (g1b_v, g2b_v, wb_v, rowsb_v, semgb, semsb),
    }

    def g_issue(c, b):
        g1, g2, w, rows, semg, _ = bufs[b]
        i0 = e0f_v.at[pl.ds(c * CHUNK, CHUNK)]
        i1 = e1f_v.at[pl.ds(c * CHUNK, CHUNK)]
        pltpu.async_copy(s1_hbm.at[i0], g1, semg)
        pltpu.async_copy(s2_hbm.at[i1], g2, semg)
        pltpu.async_copy(h2_hbm.at[i1], rows, semg)

    def g_wait(b):
        g1, g2, w, rows, semg, _ = bufs[b]
        i0 = e0f_v.at[pl.ds(0, CHUNK)]
        pltpu.make_async_copy(s1_hbm.at[i0], g1, semg).wait()
        pltpu.make_async_copy(s2_hbm.at[i0], g2, semg).wait()
        pltpu.make_async_copy(h2_hbm.at[i0], rows, semg).wait()

    def s_issue(c, b):
        g1, g2, w, rows, _, sems = bufs[b]
        i0 = e0f_v.at[pl.ds(c * CHUNK, CHUNK)]
        pltpu.async_copy(rows, acc_sh.at[i0], sems, add=True)
        pltpu.async_copy(w, rs_sh.at[i0], sems, add=True)

    def s_wait(b):
        g1, g2, w, rows, _, sems = bufs[b]
        i0 = e0f_v.at[pl.ds(0, CHUNK)]
        pltpu.make_async_copy(rows, acc_sh.at[i0], sems).wait()
        pltpu.make_async_copy(w, rs_sh.at[i0], sems).wait()

    def compute_scale(b):
        g1, g2, w, rows, _, _ = bufs[b]
        for j in range(CHUNK // L):
            x = g1[pl.ds(j * L, L)] + g2[pl.ds(j * L, L)]
            w[pl.ds(j * L, L)] = jnp.exp(-jnp.maximum(x, NEG_SLOPE * x))

        def srow(g, carry2):
            wgrp = w[pl.ds(g * L, L)]
            for u in range(L):
                i = g * L + u
                wv = wgrp[u]
                for j in range(D // L):
                    rows[i, pl.ds(j * L, L)] = rows[i, pl.ds(j * L, L)] * wv
            return carry2

        lax.fori_loop(0, CHUNK // L, srow, 0)

    g_issue(0, 0)
    s_issue(0, 1)

    def chunk_pair(c2, carry):
        cc = 2 * c2
        g_wait(0)
        s_wait(1)
        g_issue(cc + 1, 1)
        compute_scale(0)
        s_issue(cc, 0)
        g_wait(1)
        s_wait(0)
        g_issue(cc + 2, 0)
        compute_scale(1)
        s_issue(cc + 1, 1)
        return carry

    lax.fori_loop(0, (NCHUNK - 1) // 2, chunk_pair, 0)
    g_wait(0)
    s_wait(1)
    compute_scale(0)
    s_issue(NCHUNK - 1, 0)
    s_wait(0)
    plsc.subcore_barrier()

    pltpu.sync_copy(acc_sh.at[pl.ds(n0, COVER)],
                    part_hbm.at[cid, pl.ds(n0, COVER)])
    pltpu.sync_copy(rs_sh.at[pl.ds(n0, COVER)],
                    rsum_hbm.at[cid, pl.ds(n0, COVER)])


_sc_call = pl.kernel(
    _sc_body,
    out_type=(
        jax.ShapeDtypeStruct((NC, N, D), jnp.float32),
        jax.ShapeDtypeStruct((NC, N), jnp.float32),
    ),
    mesh=plsc.VectorSubcoreMesh(core_axis_name="c", subcore_axis_name="s",
                                num_cores=NC, num_subcores=NS),
    compiler_params=pltpu.CompilerParams(use_tc_tiling_on_sc=False,
                                         needs_layout_passes=False),
    scratch_types=[
        pltpu.VMEM((EPW,), jnp.int32),
        pltpu.VMEM((EPW,), jnp.int32),
        pltpu.VMEM((CHUNK,), jnp.float32),
        pltpu.VMEM((CHUNK,), jnp.float32),
        pltpu.VMEM((CHUNK,), jnp.float32),
        pltpu.VMEM((CHUNK,), jnp.float32),
        pltpu.VMEM((CHUNK,), jnp.float32),
        pltpu.VMEM((CHUNK,), jnp.float32),
        pltpu.VMEM((CHUNK, D), jnp.float32),
        pltpu.VMEM((CHUNK, D), jnp.float32),
        pltpu.VMEM_SHARED((N, D), jnp.float32),
        pltpu.VMEM_SHARED((N,), jnp.float32),
        pltpu.SemaphoreType.DMA,
        pltpu.SemaphoreType.DMA,
        pltpu.SemaphoreType.DMA,
        pltpu.SemaphoreType.DMA,
    ],
)


def _comb_body(h1_ref, part_ref, rsum_ref, o_ref):
    acc = part_ref[0] + part_ref[1]
    rs = rsum_ref[0] + rsum_ref[1]
    denom = jnp.where(rs == 0.0, 1e-12, rs)
    h = (h1_ref[...] * rs + acc) / denom
    o_ref[...] = jnp.where(h > 0.0, h, jnp.exp(h) - 1.0)


_comb_call = pl.pallas_call(
    _comb_body,
    out_shape=jax.ShapeDtypeStruct((N, D), jnp.float32),
)


def kernel(input_, edge, W, a):
    edge = edge.astype(jnp.int32)
    h1, h2, s1, s2 = _mm_call(input_, W, a)
    part, rsum = _sc_call(edge[0], edge[1], s1.reshape(N), s2.reshape(N), h2)
    return _comb_call(h1, part, rsum.reshape(NC, N, 1))
